# baseline (speedup 1.0000x reference)
import jax
import jax.numpy as jnp
import numpy as np
from jax import lax
from jax.experimental import pallas as pl
from jax.experimental.pallas import tpu as pltpu
from jax.experimental.pallas import tpu_sc as plsc

N = 10000
E = 320000
D = 128
DK = 32
NB = 10
MAXR = 3.0

NC = 2
NS = 16
NW = NC * NS
L = 16

NP = 10240
EPW = E // NW
GC = 400
CB = 2000
SL = NP // NS
RT = N // NS
BE = 2560


def _sc_mesh():
    return plsc.VectorSubcoreMesh(
        core_axis_name="c", subcore_axis_name="s",
        num_cores=NC, num_subcores=NS)


_SC_PARAMS = pltpu.CompilerParams(use_tc_tiling_on_sc=False,
                                  needs_layout_passes=False)


def _wid():
    return lax.axis_index("s") * NC + lax.axis_index("c")


def _proj_body(x_ref, pos_ref, wq_ref, xa_ref, qa_ref):
    xa_ref[:, :D] = x_ref[...]
    xa_ref[:, D:] = pos_ref[...]
    qa_ref[:, :DK] = jnp.dot(x_ref[...], wq_ref[...],
                             preferred_element_type=jnp.float32)
    qa_ref[:, DK:] = pos_ref[...]


def _proj_stage(x, pospad, Wq):
    return pl.pallas_call(
        _proj_body,
        out_shape=[jax.ShapeDtypeStruct((N, DXA), jnp.float32),
                   jax.ShapeDtypeStruct((N, DQA), jnp.float32)],
    )(x, pospad, Wq)


DXA = D + 16
DQA = DK + 16


RB = BE // 128
NL = E // 128


def _edge_body(xsa_ref, qa_ref, wk1t_ref, wv1t_ref, wklint_ref,
               wk2t_ref, wv2_ref, wvlin_ref, se_ref, vw_ref):
    psT = jnp.transpose(xsa_ref[:, D:DXA], (1, 0)).reshape(16, RB, 128)
    pdT = jnp.transpose(qa_ref[:, DK:DQA], (1, 0)).reshape(16, RB, 128)
    ev = pdT - psT
    r2 = jnp.sum(ev * ev, axis=0)
    r = jnp.sqrt(r2 + 1e-9)
    width = MAXR / NB
    rbf = jnp.stack([
        jnp.exp(-(((r - (MAXR / (NB - 1)) * k) / width) ** 2))
        for k in range(NB)
    ]) * np.sqrt(NB)
    rbf2 = rbf.reshape(NB, BE)
    at = jax.nn.silu(jnp.dot(wk1t_ref[...], rbf2,
                             preferred_element_type=jnp.float32))
    bt = jax.nn.silu(jnp.dot(wv1t_ref[...], rbf2,
                             preferred_element_type=jnp.float32))
    cut = 0.5 * (jnp.cos(jnp.pi * jnp.clip(r / MAXR, 0.0, 1.0)) + 1.0)

    u = jnp.dot(qa_ref[:, :DK], wklint_ref[...],
                preferred_element_type=jnp.float32)
    xs = xsa_ref[:, :D]
    p = jnp.dot(u * xs, wk2t_ref[...],
                preferred_element_type=jnp.float32)
    pT = jnp.transpose(p, (1, 0)).reshape(16, RB, 128)
    at3 = at.reshape(16, RB, 128)
    acc = at3[0] * pT[0]
    for j in range(1, 16):
        acc = acc + at3[j] * pT[j]
    se = jnp.sqrt(cut * jnp.exp(acc))
    se_ref[...] = se.reshape(1, BE)

    b = jnp.transpose(bt, (1, 0))
    wv = jnp.dot(b, wv2_ref[...], preferred_element_type=jnp.float32)
    v = jnp.dot(xs * wv, wvlin_ref[...],
                preferred_element_type=jnp.float32)
    vw_ref[...] = v * jnp.transpose(se.reshape(1, BE), (1, 0))


def _edge_stage(xsa, qa, Wk1_T, Wv1_T, Wk_lin_Ts, Wk2_T, Wv2, Wv_lin):
    nblk = E // BE
    full = lambda a: pl.BlockSpec(a.shape, lambda i: (0, 0))
    return pl.pallas_call(
        _edge_body,
        grid=(nblk,),
        in_specs=[pl.BlockSpec((BE, DXA), lambda i: (i, 0)),
                  pl.BlockSpec((BE, DQA), lambda i: (i, 0)),
                  full(Wk1_T), full(Wv1_T), full(Wk_lin_Ts), full(Wk2_T),
                  full(Wv2), full(Wv_lin)],
        out_specs=[pl.BlockSpec((1, BE), lambda i: (0, i)),
                   pl.BlockSpec((BE, D), lambda i: (i, 0))],
        out_shape=[jax.ShapeDtypeStruct((1, E), jnp.float32),
                   jax.ShapeDtypeStruct((E, D), jnp.float32)],
    )(xsa, qa, Wk1_T, Wv1_T, Wk_lin_Ts, Wk2_T, Wv2, Wv_lin)


def _finish_body(p_ref, ztp_ref, o_ref):
    zsum = jnp.sum(ztp_ref[...], axis=1, keepdims=True) + 1e-9
    o_ref[...] = (p_ref[0] + p_ref[1]) * lax.rsqrt(zsum)


def _finish_stage(outp, zpartT):
    nblk = 5
    rows = N // nblk
    return pl.pallas_call(
        _finish_body,
        grid=(nblk,),
        in_specs=[pl.BlockSpec((NC, rows, D), lambda i: (0, i, 0)),
                  pl.BlockSpec((rows, NW), lambda i: (i, 0))],
        out_specs=pl.BlockSpec((rows, D), lambda i: (i, 0)),
        out_shape=jax.ShapeDtypeStruct((N, D), jnp.float32),
    )(outp, zpartT)


GCG = 200


def _gather_body(xa_hbm, qa_hbm, src_hbm, dst_hbm,
                 xsa_out, qa_out,
                 idx_s, idx_d, xs_buf0, xs_buf1, q_buf0, q_buf1,
                 gsem, wsem0, wsem1):
    wid = _wid()
    xsb = (xs_buf0, xs_buf1)
    qb = (q_buf0, q_buf1)
    wsem = (wsem0, wsem1)

    def fire(cidx, b):
        base = wid * EPW + cidx * GCG
        pltpu.sync_copy(src_hbm.at[pl.ds(base, GCG)], idx_s)
        pltpu.sync_copy(dst_hbm.at[pl.ds(base, GCG)], idx_d)
        pltpu.async_copy(xa_hbm.at[idx_s], xsb[b], gsem).wait()
        pltpu.async_copy(qa_hbm.at[idx_d], qb[b], gsem).wait()
        pltpu.make_async_copy(
            xsb[b], xsa_out.at[pl.ds(base, GCG), :], wsem[b]).start()
        pltpu.make_async_copy(
            qb[b], qa_out.at[pl.ds(base, GCG), :], wsem[b]).start()

    def drain(b):
        base0 = wid * EPW
        pltpu.make_async_copy(
            xsb[b], xsa_out.at[pl.ds(base0, GCG), :], wsem[b]).wait()
        pltpu.make_async_copy(
            qb[b], qa_out.at[pl.ds(base0, GCG), :], wsem[b]).wait()

    fire(0, 0)
    fire(1, 1)

    def pair(cp, carry):
        for b in (0, 1):
            drain(b)
            fire(2 * cp + b, b)
        return carry

    lax.fori_loop(1, EPW // (2 * GCG), pair, 0)
    drain(0)
    drain(1)


def _gather_stage(xa, qa_table, src, dst):
    kern = pl.kernel(
        _gather_body,
        out_type=[jax.ShapeDtypeStruct((E, DXA), jnp.float32),
                  jax.ShapeDtypeStruct((E, DQA), jnp.float32)],
        mesh=_sc_mesh(),
        compiler_params=_SC_PARAMS,
        scratch_types=[pltpu.VMEM((GCG,), jnp.int32),
                       pltpu.VMEM((GCG,), jnp.int32),
                       pltpu.VMEM((GCG, DXA), jnp.float32),
                       pltpu.VMEM((GCG, DXA), jnp.float32),
                       pltpu.VMEM((GCG, DQA), jnp.float32),
                       pltpu.VMEM((GCG, DQA), jnp.float32),
                       pltpu.SemaphoreType.DMA,
                       pltpu.SemaphoreType.DMA,
                       pltpu.SemaphoreType.DMA],
    )
    return kern(xa, qa_table, src, dst)


GC2 = 200
ZR = 25


def _scatter_body(vw_hbm, se_hbm, dst_hbm, outp_out, zpart_out,
                  acc, vbuf, idxb, z_loc, seb, dstb):
    cid = lax.axis_index("c")
    sid = lax.axis_index("s")
    wid = sid * NC + cid

    def zinit(i, c):
        z_loc[pl.ds(i * L, L)] = jnp.zeros((L,), jnp.float32)
        return c

    lax.fori_loop(0, NP // L, zinit, 0)

    def zloop(r, c):
        for k in range(D // L):
            vbuf[r, pl.ds(k * L, L)] = jnp.zeros((L,), jnp.float32)
        return c

    lax.fori_loop(0, ZR, zloop, 0)

    def zcopy(t, c):
        pltpu.sync_copy(vbuf.at[pl.ds(0, ZR), :],
                        acc.at[pl.ds(sid * RT + t * ZR, ZR), :])
        return c

    lax.fori_loop(0, RT // ZR, zcopy, 0)
    plsc.subcore_barrier()

    def zchunk(ci, carry):
        base = wid * EPW + ci * CB
        pltpu.sync_copy(se_hbm.at[pl.ds(base, CB)], seb)
        pltpu.sync_copy(dst_hbm.at[pl.ds(base, CB)], dstb)

        def inner(j, c2):
            sl = pl.ds(j * L, L)
            sev = seb[sl]
            plsc.addupdate_scatter(z_loc, [dstb[sl]], sev * sev)
            return c2

        lax.fori_loop(0, CB // L, inner, 0)
        return carry

    lax.fori_loop(0, EPW // CB, zchunk, 0)

    def chunk(ci, carry):
        base = wid * EPW + ci * GC2
        pltpu.sync_copy(vw_hbm.at[pl.ds(base, GC2), :], vbuf)
        pltpu.sync_copy(dst_hbm.at[pl.ds(base, GC2)], idxb)
        pltpu.sync_copy(vbuf, acc.at[idxb], add=True)
        return carry

    lax.fori_loop(0, EPW // GC2, chunk, 0)
    plsc.subcore_barrier()
    pltpu.sync_copy(acc.at[pl.ds(sid * RT, RT), :],
                    outp_out.at[cid, pl.ds(sid * RT, RT), :])
    pltpu.sync_copy(z_loc, zpart_out.at[wid])


def _scatter_stage(vw, se, dst):
    kern = pl.kernel(
        _scatter_body,
        out_type=[jax.ShapeDtypeStruct((NC, N, D), jnp.float32),
                  jax.ShapeDtypeStruct((NW, NP), jnp.float32)],
        mesh=_sc_mesh(),
        compiler_params=_SC_PARAMS,
        scratch_types=[pltpu.VMEM_SHARED((N, D), jnp.float32),
                       pltpu.VMEM((GC2, D), jnp.float32),
                       pltpu.VMEM((GC2,), jnp.int32),
                       pltpu.VMEM((NP,), jnp.float32),
                       pltpu.VMEM((CB,), jnp.float32),
                       pltpu.VMEM((CB,), jnp.int32)],
    )
    return kern(vw, se, dst)


def kernel(x, pos, edge_index, Wq, Wk_lin, Wv_lin, Wk1, Wk2, Wv1, Wv2):
    src = edge_index[0]
    dst = edge_index[1]
    pospad = jnp.pad(pos, ((0, 0), (0, 13)))
    xa, qa_table = _proj_stage(x, pospad, Wq)
    xsa, qa = _gather_stage(xa, qa_table, src, dst)
    se, vw = _edge_stage(xsa, qa, Wk1.T, Wv1.T,
                         Wk_lin.T * (1.0 / np.sqrt(DK)), Wk2.T, Wv2, Wv_lin)
    outp, zpart = _scatter_stage(vw, se.reshape(E), dst)
    return _finish_stage(outp, zpart.T[:N])
